# baseline (speedup 1.0000x reference)
import functools

import jax
import jax.numpy as jnp
from jax import lax
from jax.experimental import pallas as pl
from jax.experimental.pallas import tpu as pltpu
from jax.experimental.pallas import tpu_sc as plsc

N = 10000
H = 128
EPS = 1e-5

NC = 2
NS = 16
NW = NC * NS
CHUNK = 128
TCH = 2560
CH0 = 48
CH1 = (TCH - NS * CH0) // NS
GCH = 16
NBUF = 2
DCH = TCH // NW
NP = 10240
RPT = NP // NS
E_PAD = TCH * CHUNK

_mesh = plsc.VectorSubcoreMesh(core_axis_name="c", subcore_axis_name="s")


@functools.partial(
    pl.kernel,
    out_type=jax.ShapeDtypeStruct((NC, NP), jnp.float32),
    mesh=_mesh,
    scratch_types=[
        pltpu.VMEM((DCH, CHUNK), jnp.int32),
        pltpu.VMEM((CHUNK,), jnp.int32),
        pltpu.VMEM((CHUNK,), jnp.float32),
        pltpu.VMEM((RPT,), jnp.float32),
        pltpu.VMEM_SHARED((NP,), jnp.float32),
        pltpu.SemaphoreType.DMA,
    ],
)
def _deg_kernel(dst_hbm, deg_out, idx_all, drain_v, ones_v, stage_v, deg_sh,
                sem):
    c = lax.axis_index("c")
    s = lax.axis_index("s")
    wid = c * NS + s

    def fill_ones(i, carry):
        ones_v[pl.ds(i * 16, 16)] = jnp.ones((16,), jnp.float32)
        return carry

    lax.fori_loop(0, CHUNK // 16, fill_ones, 0)

    def fill_zeros(i, carry):
        stage_v[pl.ds(i * 16, 16)] = jnp.zeros((16,), jnp.float32)
        return carry

    lax.fori_loop(0, RPT // 16, fill_zeros, 0)

    pltpu.sync_copy(dst_hbm.at[pl.ds(pl.multiple_of(wid * DCH, 8), DCH)],
                    idx_all)
    r0 = s * RPT
    pltpu.sync_copy(stage_v, deg_sh.at[pl.ds(r0, RPT)])
    plsc.subcore_barrier()

    def fire(j, carry):
        pltpu.async_copy(ones_v, deg_sh.at[idx_all.at[j]], sem, add=True)
        return carry

    lax.fori_loop(0, DCH, fire, 0)

    def drain(j, carry):
        pltpu.make_async_copy(dst_hbm.at[0], drain_v, sem).wait()
        return carry

    lax.fori_loop(0, DCH, drain, 0)
    plsc.subcore_barrier()
    pltpu.sync_copy(deg_sh.at[pl.ds(r0, RPT)], deg_out.at[c, pl.ds(r0, RPT)])


@functools.partial(
    pl.kernel,
    out_type=jax.ShapeDtypeStruct((NC, NP, H), jnp.float32),
    mesh=_mesh,
    scratch_types=[
        pltpu.VMEM((GCH, CHUNK), jnp.int32),
        pltpu.VMEM((GCH, CHUNK), jnp.int32),
        pltpu.VMEM((CHUNK, H), jnp.float32),
        pltpu.VMEM((CHUNK, H), jnp.float32),
        pltpu.SemaphoreType.DMA,
        pltpu.SemaphoreType.DMA,
        pltpu.VMEM_SHARED((NP, H), jnp.float32),
    ],
)
def _scatter_kernel(g_hbm, src_hbm, dst_hbm, acc_out,
                    sidx_g, didx_g, rows0, rows1, gsem0, gsem1, acc_sh):
    c = lax.axis_index("c")
    s = lax.axis_index("s")
    rows = [rows0, rows1]
    gsem = [gsem0, gsem1]

    def fill_zeros(i, carry):
        rows[0][i >> 3, pl.ds((i & 7) * 16, 16)] = jnp.zeros((16,),
                                                             jnp.float32)
        return carry

    lax.fori_loop(0, CHUNK * (H // 16), fill_zeros, 0)

    r0 = s * RPT
    for t in range(RPT // CHUNK):
        pltpu.sync_copy(rows[0], acc_sh.at[pl.ds(r0 + t * CHUNK, CHUNK)])
    plsc.subcore_barrier()

    base_ch = lax.select(c == 0, s * CH0, NS * CH0 + s * CH1)
    ngrp = lax.select(c == 0, CH0 // GCH, CH1 // GCH)

    def group_body(gi, carry):
        off = pl.multiple_of(base_ch + gi * GCH, 8)
        pltpu.sync_copy(src_hbm.at[pl.ds(off, GCH)], sidx_g)
        pltpu.sync_copy(dst_hbm.at[pl.ds(off, GCH)], didx_g)
        for b in range(NBUF):
            pltpu.async_copy(g_hbm.at[sidx_g.at[b]], rows[b], gsem[b])

        def inner(i, carry2):
            for b in range(NBUF):
                lch = i * NBUF + b
                pltpu.make_async_copy(g_hbm.at[sidx_g.at[lch]], rows[b],
                                      gsem[b]).wait()
                pltpu.sync_copy(rows[b], acc_sh.at[didx_g.at[lch]], add=True)

                @pl.when(lch + NBUF < GCH)
                def _():
                    pltpu.async_copy(g_hbm.at[sidx_g.at[lch + NBUF]],
                                     rows[b], gsem[b])

            return carry2

        lax.fori_loop(0, GCH // NBUF, inner, 0)
        return carry

    lax.fori_loop(0, ngrp, group_body, 0)
    plsc.subcore_barrier()
    pltpu.sync_copy(acc_sh.at[pl.ds(r0, RPT)],
                    acc_out.at[c, pl.ds(r0, RPT)])


def _tc1_body(x_ref, gam_ref, bet_ref, w_ref, deg_ref, g_ref, dinv_ref):
    xr = x_ref[...]
    mean = jnp.mean(xr, axis=0, keepdims=True)
    xc = xr - mean
    var = jnp.mean(xc * xc, axis=0, keepdims=True)
    xn = xc * lax.rsqrt(var + EPS) * gam_ref[...] + bet_ref[...]
    h = jnp.dot(xn, w_ref[...], preferred_element_type=jnp.float32)
    dinv = lax.rsqrt(deg_ref[0] + deg_ref[1] + 1.0)
    dinv_ref[...] = dinv
    g_ref[0:N, :] = h * dinv[0:N]
    g_ref[N:NP, :] = jnp.zeros((NP - N, H), jnp.float32)


_tc1 = pl.pallas_call(
    _tc1_body,
    out_shape=[
        jax.ShapeDtypeStruct((NP, H), jnp.float32),
        jax.ShapeDtypeStruct((NP, 1), jnp.float32),
    ],
)


def _tc2_body(x_ref, acc_ref, g_ref, dinv_ref, b_ref, o_ref):
    a = acc_ref[0, 0:N, :] + acc_ref[1, 0:N, :] + g_ref[0:N, :]
    conv = a * dinv_ref[0:N] + b_ref[...]
    o_ref[...] = x_ref[...] + jnp.maximum(conv, 0.0)


_tc2 = pl.pallas_call(
    _tc2_body,
    out_shape=jax.ShapeDtypeStruct((N, H), jnp.float32),
)


def kernel(x, edge_index, bn_gamma, bn_beta, W, b):
    src = edge_index[0].astype(jnp.int32)
    dst = edge_index[1].astype(jnp.int32)
    pad = E_PAD - src.shape[0]
    src = jnp.concatenate([src, jnp.full((pad,), N, jnp.int32)])
    dst = jnp.concatenate([dst, jnp.full((pad,), N, jnp.int32)])
    src = src.reshape(TCH, CHUNK)
    dst = dst.reshape(TCH, CHUNK)

    deg = _deg_kernel(dst)
    g, dinv = _tc1(x, bn_gamma.reshape(1, H), bn_beta.reshape(1, H), W,
                   deg.reshape(NC, NP, 1))
    acc = _scatter_kernel(g, src, dst)
    return _tc2(x, acc, g, dinv, b.reshape(1, H))


# speedup vs baseline: 1.0373x; 1.0373x over previous
import functools

import jax
import jax.numpy as jnp
from jax import lax
from jax.experimental import pallas as pl
from jax.experimental.pallas import tpu as pltpu
from jax.experimental.pallas import tpu_sc as plsc

N = 10000
H = 128
EPS = 1e-5

NC = 2
NS = 16
NW = NC * NS
CHUNK = 128
TCH = 2560
CH0 = 80
CH1 = (TCH - NS * CH0) // NS
GCH = 16
NBUF = 2
DCH = TCH // NW
NP = 10240
RPT = NP // NS
E_PAD = TCH * CHUNK

_mesh = plsc.VectorSubcoreMesh(core_axis_name="c", subcore_axis_name="s")


@functools.partial(
    pl.kernel,
    out_type=jax.ShapeDtypeStruct((NC, NP), jnp.float32),
    mesh=_mesh,
    scratch_types=[
        pltpu.VMEM((DCH, CHUNK), jnp.int32),
        pltpu.VMEM((CHUNK,), jnp.int32),
        pltpu.VMEM((CHUNK,), jnp.float32),
        pltpu.VMEM((RPT,), jnp.float32),
        pltpu.VMEM_SHARED((NP,), jnp.float32),
        pltpu.SemaphoreType.DMA,
    ],
)
def _deg_kernel(dst_hbm, deg_out, idx_all, drain_v, ones_v, stage_v, deg_sh,
                sem):
    c = lax.axis_index("c")
    s = lax.axis_index("s")
    wid = c * NS + s

    def fill_ones(i, carry):
        ones_v[pl.ds(i * 16, 16)] = jnp.ones((16,), jnp.float32)
        return carry

    lax.fori_loop(0, CHUNK // 16, fill_ones, 0)

    def fill_zeros(i, carry):
        stage_v[pl.ds(i * 16, 16)] = jnp.zeros((16,), jnp.float32)
        return carry

    lax.fori_loop(0, RPT // 16, fill_zeros, 0)

    pltpu.sync_copy(dst_hbm.at[pl.ds(pl.multiple_of(wid * DCH, 8), DCH)],
                    idx_all)
    r0 = s * RPT
    pltpu.sync_copy(stage_v, deg_sh.at[pl.ds(r0, RPT)])
    plsc.subcore_barrier()

    def fire(j, carry):
        pltpu.async_copy(ones_v, deg_sh.at[idx_all.at[j]], sem, add=True)
        return carry

    lax.fori_loop(0, DCH, fire, 0)

    def drain(j, carry):
        pltpu.make_async_copy(dst_hbm.at[0], drain_v, sem).wait()
        return carry

    lax.fori_loop(0, DCH, drain, 0)
    plsc.subcore_barrier()
    pltpu.sync_copy(deg_sh.at[pl.ds(r0, RPT)], deg_out.at[c, pl.ds(r0, RPT)])


@functools.partial(
    pl.kernel,
    out_type=jax.ShapeDtypeStruct((NC, NP, H), jnp.float32),
    mesh=_mesh,
    scratch_types=[
        pltpu.VMEM((GCH, CHUNK), jnp.int32),
        pltpu.VMEM((GCH, CHUNK), jnp.int32),
        pltpu.VMEM((CHUNK, H), jnp.float32),
        pltpu.VMEM((CHUNK, H), jnp.float32),
        pltpu.SemaphoreType.DMA,
        pltpu.SemaphoreType.DMA,
        pltpu.VMEM_SHARED((NP, H), jnp.float32),
    ],
)
def _scatter_kernel(g_hbm, src_hbm, dst_hbm, acc_out,
                    sidx_g, didx_g, rows0, rows1, gsem0, gsem1, acc_sh):
    c = lax.axis_index("c")
    s = lax.axis_index("s")
    rows = [rows0, rows1]
    gsem = [gsem0, gsem1]

    def fill_zeros(i, carry):
        rows[0][i >> 3, pl.ds((i & 7) * 16, 16)] = jnp.zeros((16,),
                                                             jnp.float32)
        return carry

    lax.fori_loop(0, CHUNK * (H // 16), fill_zeros, 0)

    r0 = s * RPT
    for t in range(RPT // CHUNK):
        pltpu.sync_copy(rows[0], acc_sh.at[pl.ds(r0 + t * CHUNK, CHUNK)])
    plsc.subcore_barrier()

    base_ch = lax.select(c == 0, s * CH0, NS * CH0 + s * CH1)
    ngrp = lax.select(c == 0, CH0 // GCH, CH1 // GCH)

    def group_body(gi, carry):
        off = pl.multiple_of(base_ch + gi * GCH, 8)
        pltpu.sync_copy(src_hbm.at[pl.ds(off, GCH)], sidx_g)
        pltpu.sync_copy(dst_hbm.at[pl.ds(off, GCH)], didx_g)
        for b in range(NBUF):
            pltpu.async_copy(g_hbm.at[sidx_g.at[b]], rows[b], gsem[b])

        def inner(i, carry2):
            for b in range(NBUF):
                lch = i * NBUF + b
                pltpu.make_async_copy(g_hbm.at[sidx_g.at[lch]], rows[b],
                                      gsem[b]).wait()
                pltpu.sync_copy(rows[b], acc_sh.at[didx_g.at[lch]], add=True)

                @pl.when(lch + NBUF < GCH)
                def _():
                    pltpu.async_copy(g_hbm.at[sidx_g.at[lch + NBUF]],
                                     rows[b], gsem[b])

            return carry2

        lax.fori_loop(0, GCH // NBUF, inner, 0)
        return carry

    lax.fori_loop(0, ngrp, group_body, 0)
    plsc.subcore_barrier()
    pltpu.sync_copy(acc_sh.at[pl.ds(r0, RPT)],
                    acc_out.at[c, pl.ds(r0, RPT)])


def _tc1_body(x_ref, gam_ref, bet_ref, w_ref, deg_ref, g_ref, dinv_ref):
    xr = x_ref[...]
    mean = jnp.mean(xr, axis=0, keepdims=True)
    xc = xr - mean
    var = jnp.mean(xc * xc, axis=0, keepdims=True)
    xn = xc * lax.rsqrt(var + EPS) * gam_ref[...] + bet_ref[...]
    h = jnp.dot(xn, w_ref[...], preferred_element_type=jnp.float32)
    dinv = lax.rsqrt(deg_ref[0] + deg_ref[1] + 1.0)
    dinv_ref[...] = dinv
    g_ref[0:N, :] = h * dinv[0:N]
    g_ref[N:NP, :] = jnp.zeros((NP - N, H), jnp.float32)


_tc1 = pl.pallas_call(
    _tc1_body,
    out_shape=[
        jax.ShapeDtypeStruct((NP, H), jnp.float32),
        jax.ShapeDtypeStruct((NP, 1), jnp.float32),
    ],
)


def _tc2_body(x_ref, acc_ref, g_ref, dinv_ref, b_ref, o_ref):
    a = acc_ref[0, 0:N, :] + acc_ref[1, 0:N, :] + g_ref[0:N, :]
    conv = a * dinv_ref[0:N] + b_ref[...]
    o_ref[...] = x_ref[...] + jnp.maximum(conv, 0.0)


_tc2 = pl.pallas_call(
    _tc2_body,
    out_shape=jax.ShapeDtypeStruct((N, H), jnp.float32),
)


def kernel(x, edge_index, bn_gamma, bn_beta, W, b):
    src = edge_index[0].astype(jnp.int32)
    dst = edge_index[1].astype(jnp.int32)
    pad = E_PAD - src.shape[0]
    src = jnp.concatenate([src, jnp.full((pad,), N, jnp.int32)])
    dst = jnp.concatenate([dst, jnp.full((pad,), N, jnp.int32)])
    src = src.reshape(TCH, CHUNK)
    dst = dst.reshape(TCH, CHUNK)

    deg = _deg_kernel(dst)
    g, dinv = _tc1(x, bn_gamma.reshape(1, H), bn_beta.reshape(1, H), W,
                   deg.reshape(NC, NP, 1))
    acc = _scatter_kernel(g, src, dst)
    return _tc2(x, acc, g, dinv, b.reshape(1, H))


# speedup vs baseline: 1.0968x; 1.0573x over previous
import functools

import jax
import jax.numpy as jnp
from jax import lax
from jax.experimental import pallas as pl
from jax.experimental.pallas import tpu as pltpu
from jax.experimental.pallas import tpu_sc as plsc

N = 10000
H = 128
EPS = 1e-5

NC = 2
NS = 16
NW = NC * NS
CHUNK = 128
TCH = 2560
CH0 = 128
CH1 = (TCH - NS * CH0) // NS
GCH = 16
NBUF = 2
DCH = TCH // NW
NP = 10240
RPT = NP // NS
E_PAD = TCH * CHUNK

_mesh = plsc.VectorSubcoreMesh(core_axis_name="c", subcore_axis_name="s")


@functools.partial(
    pl.kernel,
    out_type=jax.ShapeDtypeStruct((NC, NP), jnp.float32),
    mesh=_mesh,
    scratch_types=[
        pltpu.VMEM((DCH, CHUNK), jnp.int32),
        pltpu.VMEM((CHUNK,), jnp.int32),
        pltpu.VMEM((CHUNK,), jnp.float32),
        pltpu.VMEM((RPT,), jnp.float32),
        pltpu.VMEM_SHARED((NP,), jnp.float32),
        pltpu.SemaphoreType.DMA,
    ],
)
def _deg_kernel(dst_hbm, deg_out, idx_all, drain_v, ones_v, stage_v, deg_sh,
                sem):
    c = lax.axis_index("c")
    s = lax.axis_index("s")
    wid = c * NS + s

    def fill_ones(i, carry):
        ones_v[pl.ds(i * 16, 16)] = jnp.ones((16,), jnp.float32)
        return carry

    lax.fori_loop(0, CHUNK // 16, fill_ones, 0)

    def fill_zeros(i, carry):
        stage_v[pl.ds(i * 16, 16)] = jnp.zeros((16,), jnp.float32)
        return carry

    lax.fori_loop(0, RPT // 16, fill_zeros, 0)

    pltpu.sync_copy(dst_hbm.at[pl.ds(pl.multiple_of(wid * DCH, 8), DCH)],
                    idx_all)
    r0 = s * RPT
    pltpu.sync_copy(stage_v, deg_sh.at[pl.ds(r0, RPT)])
    plsc.subcore_barrier()

    def fire(j, carry):
        pltpu.async_copy(ones_v, deg_sh.at[idx_all.at[j]], sem, add=True)
        return carry

    lax.fori_loop(0, DCH, fire, 0)

    def drain(j, carry):
        pltpu.make_async_copy(dst_hbm.at[0], drain_v, sem).wait()
        return carry

    lax.fori_loop(0, DCH, drain, 0)
    plsc.subcore_barrier()
    pltpu.sync_copy(deg_sh.at[pl.ds(r0, RPT)], deg_out.at[c, pl.ds(r0, RPT)])


@functools.partial(
    pl.kernel,
    out_type=jax.ShapeDtypeStruct((NC, NP, H), jnp.float32),
    mesh=_mesh,
    scratch_types=[
        pltpu.VMEM((GCH, CHUNK), jnp.int32),
        pltpu.VMEM((GCH, CHUNK), jnp.int32),
        pltpu.VMEM((CHUNK, H), jnp.float32),
        pltpu.VMEM((CHUNK, H), jnp.float32),
        pltpu.SemaphoreType.DMA,
        pltpu.SemaphoreType.DMA,
        pltpu.VMEM_SHARED((NP, H), jnp.float32),
    ],
)
def _scatter_kernel(g_hbm, src_hbm, dst_hbm, acc_out,
                    sidx_g, didx_g, rows0, rows1, gsem0, gsem1, acc_sh):
    c = lax.axis_index("c")
    s = lax.axis_index("s")
    rows = [rows0, rows1]
    gsem = [gsem0, gsem1]

    def fill_zeros(i, carry):
        rows[0][i >> 3, pl.ds((i & 7) * 16, 16)] = jnp.zeros((16,),
                                                             jnp.float32)
        return carry

    lax.fori_loop(0, CHUNK * (H // 16), fill_zeros, 0)

    r0 = s * RPT
    for t in range(RPT // CHUNK):
        pltpu.sync_copy(rows[0], acc_sh.at[pl.ds(r0 + t * CHUNK, CHUNK)])
    plsc.subcore_barrier()

    base_ch = lax.select(c == 0, s * CH0, NS * CH0 + s * CH1)
    ngrp = lax.select(c == 0, CH0 // GCH, CH1 // GCH)

    def group_body(gi, carry):
        off = pl.multiple_of(base_ch + gi * GCH, 8)
        pltpu.sync_copy(src_hbm.at[pl.ds(off, GCH)], sidx_g)
        pltpu.sync_copy(dst_hbm.at[pl.ds(off, GCH)], didx_g)
        for b in range(NBUF):
            pltpu.async_copy(g_hbm.at[sidx_g.at[b]], rows[b], gsem[b])

        def inner(i, carry2):
            for b in range(NBUF):
                lch = i * NBUF + b
                pltpu.make_async_copy(g_hbm.at[sidx_g.at[lch]], rows[b],
                                      gsem[b]).wait()
                pltpu.sync_copy(rows[b], acc_sh.at[didx_g.at[lch]], add=True)

                @pl.when(lch + NBUF < GCH)
                def _():
                    pltpu.async_copy(g_hbm.at[sidx_g.at[lch + NBUF]],
                                     rows[b], gsem[b])

            return carry2

        lax.fori_loop(0, GCH // NBUF, inner, 0)
        return carry

    lax.fori_loop(0, ngrp, group_body, 0)
    plsc.subcore_barrier()
    pltpu.sync_copy(acc_sh.at[pl.ds(r0, RPT)],
                    acc_out.at[c, pl.ds(r0, RPT)])


def _tc1_body(x_ref, gam_ref, bet_ref, w_ref, deg_ref, g_ref, dinv_ref):
    xr = x_ref[...]
    mean = jnp.mean(xr, axis=0, keepdims=True)
    xc = xr - mean
    var = jnp.mean(xc * xc, axis=0, keepdims=True)
    xn = xc * lax.rsqrt(var + EPS) * gam_ref[...] + bet_ref[...]
    h = jnp.dot(xn, w_ref[...], preferred_element_type=jnp.float32)
    dinv = lax.rsqrt(deg_ref[0] + deg_ref[1] + 1.0)
    dinv_ref[...] = dinv
    g_ref[0:N, :] = h * dinv[0:N]
    g_ref[N:NP, :] = jnp.zeros((NP - N, H), jnp.float32)


_tc1 = pl.pallas_call(
    _tc1_body,
    out_shape=[
        jax.ShapeDtypeStruct((NP, H), jnp.float32),
        jax.ShapeDtypeStruct((NP, 1), jnp.float32),
    ],
)


def _tc2_body(x_ref, acc_ref, g_ref, dinv_ref, b_ref, o_ref):
    a = acc_ref[0, 0:N, :] + acc_ref[1, 0:N, :] + g_ref[0:N, :]
    conv = a * dinv_ref[0:N] + b_ref[...]
    o_ref[...] = x_ref[...] + jnp.maximum(conv, 0.0)


_tc2 = pl.pallas_call(
    _tc2_body,
    out_shape=jax.ShapeDtypeStruct((N, H), jnp.float32),
)


def kernel(x, edge_index, bn_gamma, bn_beta, W, b):
    src = edge_index[0].astype(jnp.int32)
    dst = edge_index[1].astype(jnp.int32)
    pad = E_PAD - src.shape[0]
    src = jnp.concatenate([src, jnp.full((pad,), N, jnp.int32)])
    dst = jnp.concatenate([dst, jnp.full((pad,), N, jnp.int32)])
    src = src.reshape(TCH, CHUNK)
    dst = dst.reshape(TCH, CHUNK)

    deg = _deg_kernel(dst)
    g, dinv = _tc1(x, bn_gamma.reshape(1, H), bn_beta.reshape(1, H), W,
                   deg.reshape(NC, NP, 1))
    acc = _scatter_kernel(g, src, dst)
    return _tc2(x, acc, g, dinv, b.reshape(1, H))


# speedup vs baseline: 1.1205x; 1.0217x over previous
import functools

import jax
import jax.numpy as jnp
from jax import lax
from jax.experimental import pallas as pl
from jax.experimental.pallas import tpu as pltpu
from jax.experimental.pallas import tpu_sc as plsc

N = 10000
H = 128
EPS = 1e-5

NC = 2
NS = 16
NW = NC * NS
CHUNK = 128
TCH = 2560
CH0 = 144
CH1 = (TCH - NS * CH0) // NS
GCH = 16
NBUF = 2
DCH = TCH // NW
NP = 10240
RPT = NP // NS
E_PAD = TCH * CHUNK

_mesh = plsc.VectorSubcoreMesh(core_axis_name="c", subcore_axis_name="s")


@functools.partial(
    pl.kernel,
    out_type=jax.ShapeDtypeStruct((NC, NP), jnp.float32),
    mesh=_mesh,
    scratch_types=[
        pltpu.VMEM((DCH, CHUNK), jnp.int32),
        pltpu.VMEM((CHUNK,), jnp.int32),
        pltpu.VMEM((CHUNK,), jnp.float32),
        pltpu.VMEM((RPT,), jnp.float32),
        pltpu.VMEM_SHARED((NP,), jnp.float32),
        pltpu.SemaphoreType.DMA,
    ],
)
def _deg_kernel(dst_hbm, deg_out, idx_all, drain_v, ones_v, stage_v, deg_sh,
                sem):
    c = lax.axis_index("c")
    s = lax.axis_index("s")
    wid = c * NS + s

    def fill_ones(i, carry):
        ones_v[pl.ds(i * 16, 16)] = jnp.ones((16,), jnp.float32)
        return carry

    lax.fori_loop(0, CHUNK // 16, fill_ones, 0)

    def fill_zeros(i, carry):
        stage_v[pl.ds(i * 16, 16)] = jnp.zeros((16,), jnp.float32)
        return carry

    lax.fori_loop(0, RPT // 16, fill_zeros, 0)

    pltpu.sync_copy(dst_hbm.at[pl.ds(pl.multiple_of(wid * DCH, 8), DCH)],
                    idx_all)
    r0 = s * RPT
    pltpu.sync_copy(stage_v, deg_sh.at[pl.ds(r0, RPT)])
    plsc.subcore_barrier()

    def fire(j, carry):
        pltpu.async_copy(ones_v, deg_sh.at[idx_all.at[j]], sem, add=True)
        return carry

    lax.fori_loop(0, DCH, fire, 0)

    def drain(j, carry):
        pltpu.make_async_copy(dst_hbm.at[0], drain_v, sem).wait()
        return carry

    lax.fori_loop(0, DCH, drain, 0)
    plsc.subcore_barrier()
    pltpu.sync_copy(deg_sh.at[pl.ds(r0, RPT)], deg_out.at[c, pl.ds(r0, RPT)])


@functools.partial(
    pl.kernel,
    out_type=jax.ShapeDtypeStruct((NC, NP, H), jnp.float32),
    mesh=_mesh,
    scratch_types=[
        pltpu.VMEM((GCH, CHUNK), jnp.int32),
        pltpu.VMEM((GCH, CHUNK), jnp.int32),
        pltpu.VMEM((CHUNK, H), jnp.float32),
        pltpu.VMEM((CHUNK, H), jnp.float32),
        pltpu.SemaphoreType.DMA,
        pltpu.SemaphoreType.DMA,
        pltpu.VMEM_SHARED((NP, H), jnp.float32),
    ],
)
def _scatter_kernel(g_hbm, src_hbm, dst_hbm, acc_out,
                    sidx_g, didx_g, rows0, rows1, gsem0, gsem1, acc_sh):
    c = lax.axis_index("c")
    s = lax.axis_index("s")
    rows = [rows0, rows1]
    gsem = [gsem0, gsem1]

    def fill_zeros(i, carry):
        rows[0][i >> 3, pl.ds((i & 7) * 16, 16)] = jnp.zeros((16,),
                                                             jnp.float32)
        return carry

    lax.fori_loop(0, CHUNK * (H // 16), fill_zeros, 0)

    r0 = s * RPT
    for t in range(RPT // CHUNK):
        pltpu.sync_copy(rows[0], acc_sh.at[pl.ds(r0 + t * CHUNK, CHUNK)])
    plsc.subcore_barrier()

    base_ch = lax.select(c == 0, s * CH0, NS * CH0 + s * CH1)
    ngrp = lax.select(c == 0, CH0 // GCH, CH1 // GCH)

    def group_body(gi, carry):
        off = pl.multiple_of(base_ch + gi * GCH, 8)
        pltpu.sync_copy(src_hbm.at[pl.ds(off, GCH)], sidx_g)
        pltpu.sync_copy(dst_hbm.at[pl.ds(off, GCH)], didx_g)
        for b in range(NBUF):
            pltpu.async_copy(g_hbm.at[sidx_g.at[b]], rows[b], gsem[b])

        def inner(i, carry2):
            for b in range(NBUF):
                lch = i * NBUF + b
                pltpu.make_async_copy(g_hbm.at[sidx_g.at[lch]], rows[b],
                                      gsem[b]).wait()
                pltpu.sync_copy(rows[b], acc_sh.at[didx_g.at[lch]], add=True)

                @pl.when(lch + NBUF < GCH)
                def _():
                    pltpu.async_copy(g_hbm.at[sidx_g.at[lch + NBUF]],
                                     rows[b], gsem[b])

            return carry2

        lax.fori_loop(0, GCH // NBUF, inner, 0)
        return carry

    lax.fori_loop(0, ngrp, group_body, 0)
    plsc.subcore_barrier()
    pltpu.sync_copy(acc_sh.at[pl.ds(r0, RPT)],
                    acc_out.at[c, pl.ds(r0, RPT)])


def _tc1_body(x_ref, gam_ref, bet_ref, w_ref, deg_ref, g_ref, dinv_ref):
    xr = x_ref[...]
    mean = jnp.mean(xr, axis=0, keepdims=True)
    xc = xr - mean
    var = jnp.mean(xc * xc, axis=0, keepdims=True)
    xn = xc * lax.rsqrt(var + EPS) * gam_ref[...] + bet_ref[...]
    h = jnp.dot(xn, w_ref[...], preferred_element_type=jnp.float32)
    dinv = lax.rsqrt(deg_ref[0] + deg_ref[1] + 1.0)
    dinv_ref[...] = dinv
    g_ref[0:N, :] = h * dinv[0:N]
    g_ref[N:NP, :] = jnp.zeros((NP - N, H), jnp.float32)


_tc1 = pl.pallas_call(
    _tc1_body,
    out_shape=[
        jax.ShapeDtypeStruct((NP, H), jnp.float32),
        jax.ShapeDtypeStruct((NP, 1), jnp.float32),
    ],
)


def _tc2_body(x_ref, acc_ref, g_ref, dinv_ref, b_ref, o_ref):
    a = acc_ref[0, 0:N, :] + acc_ref[1, 0:N, :] + g_ref[0:N, :]
    conv = a * dinv_ref[0:N] + b_ref[...]
    o_ref[...] = x_ref[...] + jnp.maximum(conv, 0.0)


_tc2 = pl.pallas_call(
    _tc2_body,
    out_shape=jax.ShapeDtypeStruct((N, H), jnp.float32),
)


def kernel(x, edge_index, bn_gamma, bn_beta, W, b):
    src = edge_index[0].astype(jnp.int32)
    dst = edge_index[1].astype(jnp.int32)
    pad = E_PAD - src.shape[0]
    src = jnp.concatenate([src, jnp.full((pad,), N, jnp.int32)])
    dst = jnp.concatenate([dst, jnp.full((pad,), N, jnp.int32)])
    src = src.reshape(TCH, CHUNK)
    dst = dst.reshape(TCH, CHUNK)

    deg = _deg_kernel(dst)
    g, dinv = _tc1(x, bn_gamma.reshape(1, H), bn_beta.reshape(1, H), W,
                   deg.reshape(NC, NP, 1))
    acc = _scatter_kernel(g, src, dst)
    return _tc2(x, acc, g, dinv, b.reshape(1, H))
